# baseline (speedup 1.0000x reference)
import functools

import jax
import jax.numpy as jnp
from jax import lax
from jax.experimental import pallas as pl
from jax.experimental.pallas import tpu as pltpu
from jax.experimental.pallas import tpu_sc as plsc

B = 4
S = 4096
D = 1024
L = 16
NC = 2
NS = 16
NW = NC * NS
S_SC = 3072
S_TC = S - S_SC
POS_PER_W = S_SC // NW
R = 16
CH = R * D
NCHUNK = POS_PER_W // R
NITER = NCHUNK // 2
XRING = 4
BLK = 512

_mesh = plsc.VectorSubcoreMesh(core_axis_name="c", subcore_axis_name="s")


@functools.partial(
    pl.kernel,
    mesh=_mesh,
    out_type=jax.ShapeDtypeStruct((B, S_SC, D), jnp.float32),
    scratch_types=[
        pltpu.VMEM((XRING, R, D), jnp.float32),
        pltpu.VMEM((2, R, D), jnp.float32),
        pltpu.SemaphoreType.DMA,
        pltpu.SemaphoreType.DMA,
        pltpu.SemaphoreType.DMA,
        pltpu.SemaphoreType.DMA,
        pltpu.SemaphoreType.DMA,
        pltpu.SemaphoreType.DMA,
        pltpu.SemaphoreType.DMA,
        pltpu.SemaphoreType.DMA,
        pltpu.SemaphoreType.DMA,
        pltpu.SemaphoreType.DMA,
    ],
)
def _pos_emb_sc(x_hbm, table_hbm, out_hbm, x_v, t_v,
                in0, in1, in2, in3, out0, out1, out2, out3, ts0, ts1):
    in_sems = (in0, in1, in2, in3)
    out_sems = (out0, out1, out2, out3)
    t_sems = (ts0, ts1)
    w = lax.axis_index("s") * NC + lax.axis_index("c")
    base = w * POS_PER_W

    def start_in(c, b, slot):
        o = base + c * R
        pltpu.async_copy(x_hbm.at[b, pl.ds(o, R)], x_v.at[slot],
                         in_sems[slot])

    def wait_in(c, b, slot):
        o = base + c * R
        pltpu.make_async_copy(x_hbm.at[b, pl.ds(o, R)], x_v.at[slot],
                              in_sems[slot]).wait()

    def start_out(c, b, slot):
        o = base + c * R
        pltpu.async_copy(x_v.at[slot], out_hbm.at[b, pl.ds(o, R)],
                         out_sems[slot])

    def wait_out(c, b, slot):
        o = base + c * R
        pltpu.make_async_copy(x_v.at[slot], out_hbm.at[b, pl.ds(o, R)],
                              out_sems[slot]).wait()

    def start_t(c, ts):
        pltpu.async_copy(table_hbm.at[pl.ds(base + c * R, R)], t_v.at[ts],
                         t_sems[ts])

    def wait_t(c, ts):
        pltpu.make_async_copy(table_hbm.at[pl.ds(base + c * R, R)],
                              t_v.at[ts], t_sems[ts]).wait()

    def compute(slot, ts):
        @plsc.parallel_loop(0, R)
        def _(r):
            for jj in range(D // L):
                sl = pl.ds(jj * L, L)
                plsc.addupdate(x_v.at[slot, r, sl], t_v[ts, r, sl])

    start_t(0, 0)
    start_t(1, 1)
    start_in(0, 0, 0)
    start_in(0, 1, 1)

    def body(i, carry):
        for j in range(8):
            slot = j % 4
            ts = 0 if j < 4 else 1
            c = 2 * i + (1 if j >= 4 else 0)
            if j == 0:
                wait_t(2 * i, 0)
            if j == 4:
                wait_t(2 * i + 1, 1)
            wait_in(c, j % 4, slot)
            compute(slot, ts)
            start_out(c, j % 4, slot)

            def refill():
                wait_out(2 * i + (j - 2) // 4, (j - 2) % 4, (j - 2) % 4)
                start_in(2 * i + (j + 2) // 4, (j + 2) % 4, (j + 2) % 4)

            if j < 2:
                @pl.when(i > 0)
                def _():
                    wait_out(2 * i + (j - 2) // 4, (j - 2) % 4, (j - 2) % 4)

                start_in(2 * i + (j + 2) // 4, (j + 2) % 4, (j + 2) % 4)
            elif j >= 6:
                if_last_ok = pl.when(i < NITER - 1)(refill)
            else:
                refill()

            if j == 3:

                @pl.when(i < NITER - 1)
                def _():
                    start_t(2 * i + 2, 0)

            if j == 7:

                @pl.when(i < NITER - 1)
                def _():
                    start_t(2 * i + 3, 1)

        return carry

    lax.fori_loop(0, NITER, body, 0)

    wait_out(NCHUNK - 1, 0, 0)
    wait_out(NCHUNK - 1, 1, 1)
    wait_out(NCHUNK - 1, 2, 2)
    wait_out(NCHUNK - 1, 3, 3)


def _tc_body(x_ref, t_ref, o_ref):
    o_ref[...] = x_ref[...] + t_ref[...]


_tc_add = pl.pallas_call(
    _tc_body,
    grid=(B, S_TC // BLK),
    in_specs=[
        pl.BlockSpec((1, BLK, D), lambda b, s: (b, s + S_SC // BLK, 0)),
        pl.BlockSpec((BLK, D), lambda b, s: (s + S_SC // BLK, 0)),
    ],
    out_specs=pl.BlockSpec((1, BLK, D), lambda b, s: (b, s, 0)),
    out_shape=jax.ShapeDtypeStruct((B, S_TC, D), jnp.float32),
)


def kernel(x, table):
    sc_part = _pos_emb_sc(x, table)
    tc_part = _tc_add(x, table)
    return jnp.concatenate([sc_part, tc_part], axis=1)


# speedup vs baseline: 1.9677x; 1.9677x over previous
import jax
import jax.numpy as jnp
from jax.experimental import pallas as pl

B, S, D, BLK = 4, 4096, 1024, 512


def _tc_body(x_ref, t_ref, o_ref):
    o_ref[...] = x_ref[...] + t_ref[...]


_tc_add = pl.pallas_call(
    _tc_body,
    grid=(B, S // BLK),
    in_specs=[
        pl.BlockSpec((1, BLK, D), lambda b, s: (b, s, 0)),
        pl.BlockSpec((BLK, D), lambda b, s: (s, 0)),
    ],
    out_specs=pl.BlockSpec((1, BLK, D), lambda b, s: (b, s, 0)),
    out_shape=jax.ShapeDtypeStruct((B, S, D), jnp.float32),
)


def kernel(x, table):
    return _tc_add(x, table)
